# Initial kernel scaffold; baseline (speedup 1.0000x reference)
import jax
import jax.numpy as jnp
from jax.experimental import pallas as pl


def kernel(x, h_edge_index, W, b):
    raise NotImplementedError("write your pallas kernel here")


# speedup vs baseline: 3.7350x; 3.7350x over previous
import functools

import jax
import jax.numpy as jnp
from jax import lax
from jax.experimental import pallas as pl
from jax.experimental.pallas import tpu as pltpu
from jax.experimental.pallas import tpu_sc as plsc

N = 10000
H = 10000
E = 320000
D = 128

NC = 2
NS = 16
NW = NC * NS

K = 80
CHUNKS = E // K
CH_PER_W = CHUNKS // NW
CH_PER_T = CHUNKS // NS
RPT = 624
TAIL = H - NS * RPT

_mesh = plsc.VectorSubcoreMesh(core_axis_name="c", subcore_axis_name="s")


def _sliced_copy(sid, src_at, dst_at):
    r0 = sid * RPT
    pltpu.sync_copy(src_at(r0, RPT), dst_at(r0, RPT))

    @pl.when(sid == NS - 1)
    def _():
        pltpu.sync_copy(src_at(NS * RPT, TAIL), dst_at(NS * RPT, TAIL))


def _sc_pass():
    def body(tbl, gidx, sidx, zrows, psum, acc, gi, si, rows, sem):
        cid = lax.axis_index("c")
        sid = lax.axis_index("s")
        wid = sid * NC + cid

        _sliced_copy(sid, lambda r, n: zrows.at[pl.ds(r, n)],
                     lambda r, n: acc.at[pl.ds(r, n)])
        plsc.subcore_barrier()

        def chunk(i, carry):
            base = (wid * CH_PER_W + i) * K
            pltpu.sync_copy(sidx.at[pl.ds(base, K)], si.at[0])
            pltpu.sync_copy(gidx.at[pl.ds(base, K)], gi.at[0])
            pltpu.async_copy(tbl.at[gi.at[0]], rows, sem).wait()
            pltpu.sync_copy(rows, acc.at[si.at[0]], add=True)
            return carry

        lax.fori_loop(0, CH_PER_W, chunk, 0)
        plsc.subcore_barrier()

        _sliced_copy(sid, lambda r, n: acc.at[pl.ds(r, n)],
                     lambda r, n: psum.at[cid, pl.ds(r, n)])

    return pl.kernel(
        body,
        out_type=jax.ShapeDtypeStruct((2, H, D), jnp.float32),
        mesh=_mesh,
        scratch_types=[pltpu.VMEM_SHARED((H, D), jnp.float32),
                       pltpu.VMEM((1, K), jnp.int32),
                       pltpu.VMEM((1, K), jnp.int32),
                       pltpu.VMEM((K, D), jnp.float32),
                       pltpu.SemaphoreType.DMA])


def _sc_hist():
    def body(he, nd, zrows, ones_h, cnt_o, acc, si, ones_v, sem):
        cid = lax.axis_index("c")
        sid = lax.axis_index("s")

        _sliced_copy(sid, lambda r, n: zrows.at[pl.ds(r, n)],
                     lambda r, n: acc.at[pl.ds(r, n)])
        pltpu.sync_copy(ones_h, ones_v)
        plsc.subcore_barrier()

        def chunk(i, carry):
            base = (sid * CH_PER_T + i) * K

            @pl.when(cid == 0)
            def _():
                pltpu.sync_copy(he.at[pl.ds(base, K)], si.at[0])

            @pl.when(cid == 1)
            def _():
                pltpu.sync_copy(nd.at[pl.ds(base, K)], si.at[0])

            pltpu.sync_copy(ones_v, acc.at[si.at[0]], add=True)
            return carry

        lax.fori_loop(0, CH_PER_T, chunk, 0)
        plsc.subcore_barrier()

        _sliced_copy(sid, lambda r, n: acc.at[pl.ds(r, n)],
                     lambda r, n: cnt_o.at[cid, pl.ds(r, n)])

    return pl.kernel(
        body,
        out_type=jax.ShapeDtypeStruct((2, H, D), jnp.float32),
        mesh=_mesh,
        scratch_types=[pltpu.VMEM_SHARED((H, D), jnp.float32),
                       pltpu.VMEM((1, K), jnp.int32),
                       pltpu.VMEM((K, D), jnp.float32),
                       pltpu.SemaphoreType.DMA])


def _combine(psum, cnt):
    BLK = 1000
    G = H // BLK

    def body(p0, p1, c0, o):
        c = c0[0][:, 0:1]
        o[...] = (p0[0] + p1[0]) * (1.0 / jnp.maximum(c, 1.0))

    return pl.pallas_call(
        body,
        grid=(G,),
        in_specs=[pl.BlockSpec((1, BLK, D), lambda i: (0, i, 0)),
                  pl.BlockSpec((1, BLK, D), lambda i: (1, i, 0)),
                  pl.BlockSpec((1, BLK, D), lambda i: (0, i, 0))],
        out_specs=pl.BlockSpec((BLK, D), lambda i: (i, 0)),
        out_shape=jax.ShapeDtypeStruct((H, D), jnp.float32),
    )(psum, psum, cnt)


def _final(x, npsum, cnt, W, b2):
    BLK = 1000
    G = N // BLK

    def body(xr, p0, p1, c1, wr, br, o):
        c = c1[0][:, 0:1]
        nf = (p0[0] + p1[0]) * (1.0 / jnp.maximum(c, 1.0))
        w = wr[...]
        out = (jnp.dot(xr[...], w[:D], preferred_element_type=jnp.float32,
                       precision=lax.Precision.HIGHEST)
               + jnp.dot(nf, w[D:], preferred_element_type=jnp.float32,
                         precision=lax.Precision.HIGHEST)
               + br[...])
        s = jnp.sum(out * out, axis=1, keepdims=True)
        o[...] = out / jnp.sqrt(s)

    return pl.pallas_call(
        body,
        grid=(G,),
        in_specs=[pl.BlockSpec((BLK, D), lambda i: (i, 0)),
                  pl.BlockSpec((1, BLK, D), lambda i: (0, i, 0)),
                  pl.BlockSpec((1, BLK, D), lambda i: (1, i, 0)),
                  pl.BlockSpec((1, BLK, D), lambda i: (1, i, 0)),
                  pl.BlockSpec((2 * D, D), lambda i: (0, 0)),
                  pl.BlockSpec((1, D), lambda i: (0, 0))],
        out_specs=pl.BlockSpec((BLK, D), lambda i: (i, 0)),
        out_shape=jax.ShapeDtypeStruct((N, D), jnp.float32),
    )(x, npsum, npsum, cnt, W, b2)


def kernel(x, h_edge_index, W, b):
    he = h_edge_index[0]
    nd = h_edge_index[1]
    zrows = jnp.zeros((H, D), jnp.float32)
    ones = jnp.ones((K, D), jnp.float32)

    cnt = _sc_hist()(he, nd, zrows, ones)
    epsum = _sc_pass()(x, nd, he, zrows)
    edge_agg = _combine(epsum, cnt)
    npsum = _sc_pass()(edge_agg, he, nd, zrows)
    return _final(x, npsum, cnt, W, b.reshape(1, D))


# speedup vs baseline: 7.9642x; 2.1323x over previous
import functools

import jax
import jax.numpy as jnp
from jax import lax
from jax.experimental import pallas as pl
from jax.experimental.pallas import tpu as pltpu
from jax.experimental.pallas import tpu_sc as plsc

N = 10000
H = 10000
E = 320000
D = 128

NC = 2
NS = 16
NW = NC * NS

K = 125
CHUNKS = E // K
CPW = CHUNKS // NW
CPT = CHUNKS // NS
SEG = 16
NSEG = CPW // SEG
RPT = 624
TAIL = H - NS * RPT

_mesh = plsc.VectorSubcoreMesh(core_axis_name="c", subcore_axis_name="s")


def _sliced_copy(sid, src_at, dst_at):
    r0 = sid * RPT
    pltpu.sync_copy(src_at(r0, RPT), dst_at(r0, RPT))

    @pl.when(sid == NS - 1)
    def _():
        pltpu.sync_copy(src_at(NS * RPT, TAIL), dst_at(NS * RPT, TAIL))


def _sc_pass():
    def body(tbl, gidx2, sidx2, zrows, psum, acc, gia, sia, rows, sem):
        cid = lax.axis_index("c")
        sid = lax.axis_index("s")
        wid = sid * NC + cid

        _sliced_copy(sid, lambda r, n: zrows.at[pl.ds(r, n)],
                     lambda r, n: acc.at[pl.ds(r, n)])
        c0 = wid * CPW
        plsc.subcore_barrier()

        def seg(s, carry):
            sb = c0 + s * SEG
            pltpu.sync_copy(gidx2.at[pl.ds(sb, SEG)], gia)
            pltpu.sync_copy(sidx2.at[pl.ds(sb, SEG)], sia)
            pltpu.async_copy(tbl.at[gia.at[0]], rows.at[0], sem)

            def chunk(i, c2):
                buf = lax.rem(i, 2)

                @pl.when(i + 1 < SEG)
                def _():
                    pltpu.async_copy(tbl.at[gia.at[i + 1]],
                                     rows.at[lax.rem(i + 1, 2)], sem)

                pltpu.make_async_copy(tbl.at[gia.at[i]], rows.at[buf],
                                      sem).wait()
                pltpu.sync_copy(rows.at[buf], acc.at[sia.at[i]], add=True)
                return c2

            lax.fori_loop(0, SEG, chunk, 0)
            return carry

        lax.fori_loop(0, NSEG, seg, 0)
        plsc.subcore_barrier()

        _sliced_copy(sid, lambda r, n: acc.at[pl.ds(r, n)],
                     lambda r, n: psum.at[cid, pl.ds(r, n)])

    return pl.kernel(
        body,
        out_type=jax.ShapeDtypeStruct((2, H, D), jnp.float32),
        mesh=_mesh,
        scratch_types=[pltpu.VMEM_SHARED((H, D), jnp.float32),
                       pltpu.VMEM((SEG, K), jnp.int32),
                       pltpu.VMEM((SEG, K), jnp.int32),
                       pltpu.VMEM((2, K, D), jnp.float32),
                       pltpu.SemaphoreType.DMA])


def _sc_hist():
    def body(he2, nd2, zrows, ones_h, cnt_o, acc, sia, ones_v):
        cid = lax.axis_index("c")
        sid = lax.axis_index("s")

        _sliced_copy(sid, lambda r, n: zrows.at[pl.ds(r, n)],
                     lambda r, n: acc.at[pl.ds(r, n)])
        pltpu.sync_copy(ones_h, ones_v)
        c0 = sid * CPT

        @pl.when(cid == 0)
        def _():
            pltpu.sync_copy(he2.at[pl.ds(c0, CPT)], sia)

        @pl.when(cid == 1)
        def _():
            pltpu.sync_copy(nd2.at[pl.ds(c0, CPT)], sia)

        plsc.subcore_barrier()

        def chunk(i, carry):
            pltpu.sync_copy(ones_v, acc.at[sia.at[i]], add=True)
            return carry

        lax.fori_loop(0, CPT, chunk, 0)
        plsc.subcore_barrier()

        _sliced_copy(sid, lambda r, n: acc.at[pl.ds(r, n)],
                     lambda r, n: cnt_o.at[cid, pl.ds(r, n)])

    return pl.kernel(
        body,
        out_type=jax.ShapeDtypeStruct((2, H, D), jnp.float32),
        mesh=_mesh,
        scratch_types=[pltpu.VMEM_SHARED((H, D), jnp.float32),
                       pltpu.VMEM((CPT, K), jnp.int32),
                       pltpu.VMEM((K, D), jnp.float32)])


def _combine(psum, cnt):
    BLK = 1000
    G = H // BLK

    def body(p0, p1, c0, o):
        c = c0[0][:, 0:1]
        o[...] = (p0[0] + p1[0]) * (1.0 / jnp.maximum(c, 1.0))

    return pl.pallas_call(
        body,
        grid=(G,),
        in_specs=[pl.BlockSpec((1, BLK, D), lambda i: (0, i, 0)),
                  pl.BlockSpec((1, BLK, D), lambda i: (1, i, 0)),
                  pl.BlockSpec((1, BLK, D), lambda i: (0, i, 0))],
        out_specs=pl.BlockSpec((BLK, D), lambda i: (i, 0)),
        out_shape=jax.ShapeDtypeStruct((H, D), jnp.float32),
    )(psum, psum, cnt)


def _final(x, npsum, cnt, W, b2):
    BLK = 1000
    G = N // BLK

    def body(xr, p0, p1, c1, wr, br, o):
        c = c1[0][:, 0:1]
        nf = (p0[0] + p1[0]) * (1.0 / jnp.maximum(c, 1.0))
        w = wr[...]
        out = (jnp.dot(xr[...], w[:D], preferred_element_type=jnp.float32,
                       precision=lax.Precision.HIGHEST)
               + jnp.dot(nf, w[D:], preferred_element_type=jnp.float32,
                         precision=lax.Precision.HIGHEST)
               + br[...])
        s = jnp.sum(out * out, axis=1, keepdims=True)
        o[...] = out / jnp.sqrt(s)

    return pl.pallas_call(
        body,
        grid=(G,),
        in_specs=[pl.BlockSpec((BLK, D), lambda i: (i, 0)),
                  pl.BlockSpec((1, BLK, D), lambda i: (0, i, 0)),
                  pl.BlockSpec((1, BLK, D), lambda i: (1, i, 0)),
                  pl.BlockSpec((1, BLK, D), lambda i: (1, i, 0)),
                  pl.BlockSpec((2 * D, D), lambda i: (0, 0)),
                  pl.BlockSpec((1, D), lambda i: (0, 0))],
        out_specs=pl.BlockSpec((BLK, D), lambda i: (i, 0)),
        out_shape=jax.ShapeDtypeStruct((N, D), jnp.float32),
    )(x, npsum, npsum, cnt, W, b2)


def kernel(x, h_edge_index, W, b):
    he2 = h_edge_index[0].reshape(CHUNKS, K)
    nd2 = h_edge_index[1].reshape(CHUNKS, K)
    zrows = jnp.zeros((H, D), jnp.float32)
    ones = jnp.ones((K, D), jnp.float32)

    cnt = _sc_hist()(he2, nd2, zrows, ones)
    epsum = _sc_pass()(x, nd2, he2, zrows)
    edge_agg = _combine(epsum, cnt)
    npsum = _sc_pass()(edge_agg, he2, nd2, zrows)
    return _final(x, npsum, cnt, W, b.reshape(1, D))


# speedup vs baseline: 10.5120x; 1.3199x over previous
import functools

import jax
import jax.numpy as jnp
from jax import lax
from jax.experimental import pallas as pl
from jax.experimental.pallas import tpu as pltpu
from jax.experimental.pallas import tpu_sc as plsc

N = 10000
H = 10000
E = 320000
D = 128

NC = 2
NS = 16
NW = NC * NS

K = 125
CHUNKS = E // K
CPW = CHUNKS // NW
SEG = 16
NSEG = CPW // SEG
RPT = 624
TAIL = H - NS * RPT
HC = 10240
CRPT = HC // NS

_mesh = plsc.VectorSubcoreMesh(core_axis_name="c", subcore_axis_name="s")


def _sliced_copy(sid, src_at, dst_at):
    r0 = sid * RPT
    pltpu.sync_copy(src_at(r0, RPT), dst_at(r0, RPT))

    @pl.when(sid == NS - 1)
    def _():
        pltpu.sync_copy(src_at(NS * RPT, TAIL), dst_at(NS * RPT, TAIL))


def _sc_pass(with_counts):
    def body(*refs):
        if with_counts:
            (tbl, gidx2, sidx2, zrows, z1d, ones1,
             psum, scnt_o, gcnt_o,
             acc, scnt, gcnt, gia, sia, rows, ones_v, sem) = refs
        else:
            (tbl, gidx2, sidx2, zrows,
             psum,
             acc, gia, sia, rows, sem) = refs
        cid = lax.axis_index("c")
        sid = lax.axis_index("s")
        wid = sid * NC + cid

        _sliced_copy(sid, lambda r, n: zrows.at[pl.ds(r, n)],
                     lambda r, n: acc.at[pl.ds(r, n)])
        if with_counts:
            w0 = sid * CRPT
            pltpu.sync_copy(z1d.at[pl.ds(w0, CRPT)], scnt.at[pl.ds(w0, CRPT)])
            pltpu.sync_copy(z1d.at[pl.ds(w0, CRPT)], gcnt.at[pl.ds(w0, CRPT)])
            pltpu.sync_copy(ones1, ones_v)
        c0 = wid * CPW
        plsc.subcore_barrier()

        def seg(s, carry):
            sb = c0 + s * SEG
            pltpu.sync_copy(gidx2.at[pl.ds(sb, SEG)], gia)
            pltpu.sync_copy(sidx2.at[pl.ds(sb, SEG)], sia)
            pltpu.async_copy(tbl.at[gia.at[0]], rows.at[0], sem)

            def chunk(i, c2):
                buf = lax.rem(i, 2)

                @pl.when(i + 1 < SEG)
                def _():
                    pltpu.async_copy(tbl.at[gia.at[i + 1]],
                                     rows.at[lax.rem(i + 1, 2)], sem)

                pltpu.make_async_copy(tbl.at[gia.at[i]], rows.at[buf],
                                      sem).wait()
                pltpu.sync_copy(rows.at[buf], acc.at[sia.at[i]], add=True)
                if with_counts:
                    pltpu.sync_copy(ones_v, scnt.at[sia.at[i]], add=True)
                    pltpu.sync_copy(ones_v, gcnt.at[gia.at[i]], add=True)
                return c2

            lax.fori_loop(0, SEG, chunk, 0)
            return carry

        lax.fori_loop(0, NSEG, seg, 0)
        plsc.subcore_barrier()

        _sliced_copy(sid, lambda r, n: acc.at[pl.ds(r, n)],
                     lambda r, n: psum.at[cid, pl.ds(r, n)])
        if with_counts:
            ob = cid * HC + w0
            pltpu.sync_copy(scnt.at[pl.ds(w0, CRPT)],
                            scnt_o.at[pl.ds(ob, CRPT)])
            pltpu.sync_copy(gcnt.at[pl.ds(w0, CRPT)],
                            gcnt_o.at[pl.ds(ob, CRPT)])

    if with_counts:
        out_type = (jax.ShapeDtypeStruct((2, H, D), jnp.float32),
                    jax.ShapeDtypeStruct((2 * HC,), jnp.float32),
                    jax.ShapeDtypeStruct((2 * HC,), jnp.float32))
        scratch = [pltpu.VMEM_SHARED((H, D), jnp.float32),
                   pltpu.VMEM_SHARED((HC,), jnp.float32),
                   pltpu.VMEM_SHARED((HC,), jnp.float32),
                   pltpu.VMEM((SEG, K), jnp.int32),
                   pltpu.VMEM((SEG, K), jnp.int32),
                   pltpu.VMEM((2, K, D), jnp.float32),
                   pltpu.VMEM((K,), jnp.float32),
                   pltpu.SemaphoreType.DMA]
    else:
        out_type = jax.ShapeDtypeStruct((2, H, D), jnp.float32)
        scratch = [pltpu.VMEM_SHARED((H, D), jnp.float32),
                   pltpu.VMEM((SEG, K), jnp.int32),
                   pltpu.VMEM((SEG, K), jnp.int32),
                   pltpu.VMEM((2, K, D), jnp.float32),
                   pltpu.SemaphoreType.DMA]
    return pl.kernel(body, out_type=out_type, mesh=_mesh,
                     scratch_types=scratch)


def _combine(psum, c0v, c1v):
    BLK = 1000
    G = H // BLK

    def body(p0, p1, c0, c1, o):
        c = c0[...] + c1[...]
        o[...] = (p0[0] + p1[0]) * (1.0 / jnp.maximum(c, 1.0))

    return pl.pallas_call(
        body,
        grid=(G,),
        in_specs=[pl.BlockSpec((1, BLK, D), lambda i: (0, i, 0)),
                  pl.BlockSpec((1, BLK, D), lambda i: (1, i, 0)),
                  pl.BlockSpec((BLK, 1), lambda i: (i, 0)),
                  pl.BlockSpec((BLK, 1), lambda i: (i, 0))],
        out_specs=pl.BlockSpec((BLK, D), lambda i: (i, 0)),
        out_shape=jax.ShapeDtypeStruct((H, D), jnp.float32),
    )(psum, psum, c0v, c1v)


def _final(x, npsum, c0v, c1v, W, b2):
    BLK = 1000
    G = N // BLK

    def body(xr, p0, p1, c0, c1, wr, br, o):
        c = c0[...] + c1[...]
        nf = (p0[0] + p1[0]) * (1.0 / jnp.maximum(c, 1.0))
        w = wr[...]
        out = (jnp.dot(xr[...], w[:D], preferred_element_type=jnp.float32,
                       precision=lax.Precision.HIGHEST)
               + jnp.dot(nf, w[D:], preferred_element_type=jnp.float32,
                         precision=lax.Precision.HIGHEST)
               + br[...])
        s = jnp.sum(out * out, axis=1, keepdims=True)
        o[...] = out / jnp.sqrt(s)

    return pl.pallas_call(
        body,
        grid=(G,),
        in_specs=[pl.BlockSpec((BLK, D), lambda i: (i, 0)),
                  pl.BlockSpec((1, BLK, D), lambda i: (0, i, 0)),
                  pl.BlockSpec((1, BLK, D), lambda i: (1, i, 0)),
                  pl.BlockSpec((BLK, 1), lambda i: (i, 0)),
                  pl.BlockSpec((BLK, 1), lambda i: (i, 0)),
                  pl.BlockSpec((2 * D, D), lambda i: (0, 0)),
                  pl.BlockSpec((1, D), lambda i: (0, 0))],
        out_specs=pl.BlockSpec((BLK, D), lambda i: (i, 0)),
        out_shape=jax.ShapeDtypeStruct((N, D), jnp.float32),
    )(x, npsum, npsum, c0v, c1v, W, b2)


def kernel(x, h_edge_index, W, b):
    he2 = h_edge_index[0].reshape(CHUNKS, K)
    nd2 = h_edge_index[1].reshape(CHUNKS, K)
    zrows = jnp.zeros((H, D), jnp.float32)
    z1d = jnp.zeros((HC,), jnp.float32)
    ones1 = jnp.ones((K,), jnp.float32)

    epsum, hcnt, ncnt = _sc_pass(True)(x, nd2, he2, zrows, z1d, ones1)
    hc0 = hcnt[:H].reshape(H, 1)
    hc1 = hcnt[HC:HC + H].reshape(H, 1)
    nc0 = ncnt[:N].reshape(N, 1)
    nc1 = ncnt[HC:HC + N].reshape(N, 1)
    edge_agg = _combine(epsum, hc0, hc1)
    npsum = _sc_pass(False)(edge_agg, he2, nd2, zrows)
    return _final(x, npsum, nc0, nc1, W, b.reshape(1, D))


# speedup vs baseline: 10.8879x; 1.0358x over previous
import functools

import jax
import jax.numpy as jnp
from jax import lax
from jax.experimental import pallas as pl
from jax.experimental.pallas import tpu as pltpu
from jax.experimental.pallas import tpu_sc as plsc

N = 10000
H = 10000
E = 320000
D = 128

NC = 2
NS = 16
NW = NC * NS

K = 125
CHUNKS = E // K
CPW = CHUNKS // NW
SEG = 40
NSEG = CPW // SEG
RPT = 624
TAIL = H - NS * RPT
HC = 10240
CRPT = HC // NS

_mesh = plsc.VectorSubcoreMesh(core_axis_name="c", subcore_axis_name="s")


def _sliced_copy(sid, src_at, dst_at):
    r0 = sid * RPT
    pltpu.sync_copy(src_at(r0, RPT), dst_at(r0, RPT))

    @pl.when(sid == NS - 1)
    def _():
        pltpu.sync_copy(src_at(NS * RPT, TAIL), dst_at(NS * RPT, TAIL))


def _sc_pass(with_counts):
    def body(*refs):
        if with_counts:
            (tbl, gidx2, sidx2, zrows, z1d, ones1,
             psum, scnt_o, gcnt_o,
             acc, scnt, gcnt, gia, sia, rows, ones_v, sem) = refs
        else:
            (tbl, gidx2, sidx2, zrows,
             psum,
             acc, gia, sia, rows, sem) = refs
        cid = lax.axis_index("c")
        sid = lax.axis_index("s")
        wid = sid * NC + cid

        _sliced_copy(sid, lambda r, n: zrows.at[pl.ds(r, n)],
                     lambda r, n: acc.at[pl.ds(r, n)])
        if with_counts:
            w0 = sid * CRPT
            pltpu.sync_copy(z1d.at[pl.ds(w0, CRPT)], scnt.at[pl.ds(w0, CRPT)])
            pltpu.sync_copy(z1d.at[pl.ds(w0, CRPT)], gcnt.at[pl.ds(w0, CRPT)])
            pltpu.sync_copy(ones1, ones_v)
        c0 = wid * CPW
        plsc.subcore_barrier()

        def seg(s, carry):
            sb = c0 + s * SEG
            pltpu.sync_copy(gidx2.at[pl.ds(sb, SEG)], gia)
            pltpu.sync_copy(sidx2.at[pl.ds(sb, SEG)], sia)
            pltpu.async_copy(tbl.at[gia.at[0]], rows.at[0], sem)

            def chunk(i, c2):
                buf = lax.rem(i, 2)

                @pl.when(i + 1 < SEG)
                def _():
                    pltpu.async_copy(tbl.at[gia.at[i + 1]],
                                     rows.at[lax.rem(i + 1, 2)], sem)

                pltpu.make_async_copy(tbl.at[gia.at[i]], rows.at[buf],
                                      sem).wait()
                pltpu.sync_copy(rows.at[buf], acc.at[sia.at[i]], add=True)
                if with_counts:
                    pltpu.sync_copy(ones_v, scnt.at[sia.at[i]], add=True)
                    pltpu.sync_copy(ones_v, gcnt.at[gia.at[i]], add=True)
                return c2

            lax.fori_loop(0, SEG, chunk, 0)
            return carry

        lax.fori_loop(0, NSEG, seg, 0)
        plsc.subcore_barrier()

        _sliced_copy(sid, lambda r, n: acc.at[pl.ds(r, n)],
                     lambda r, n: psum.at[cid, pl.ds(r, n)])
        if with_counts:
            ob = cid * HC + w0
            pltpu.sync_copy(scnt.at[pl.ds(w0, CRPT)],
                            scnt_o.at[pl.ds(ob, CRPT)])
            pltpu.sync_copy(gcnt.at[pl.ds(w0, CRPT)],
                            gcnt_o.at[pl.ds(ob, CRPT)])

    if with_counts:
        out_type = (jax.ShapeDtypeStruct((2, H, D), jnp.float32),
                    jax.ShapeDtypeStruct((2 * HC,), jnp.float32),
                    jax.ShapeDtypeStruct((2 * HC,), jnp.float32))
        scratch = [pltpu.VMEM_SHARED((H, D), jnp.float32),
                   pltpu.VMEM_SHARED((HC,), jnp.float32),
                   pltpu.VMEM_SHARED((HC,), jnp.float32),
                   pltpu.VMEM((SEG, K), jnp.int32),
                   pltpu.VMEM((SEG, K), jnp.int32),
                   pltpu.VMEM((2, K, D), jnp.float32),
                   pltpu.VMEM((K,), jnp.float32),
                   pltpu.SemaphoreType.DMA]
    else:
        out_type = jax.ShapeDtypeStruct((2, H, D), jnp.float32)
        scratch = [pltpu.VMEM_SHARED((H, D), jnp.float32),
                   pltpu.VMEM((SEG, K), jnp.int32),
                   pltpu.VMEM((SEG, K), jnp.int32),
                   pltpu.VMEM((2, K, D), jnp.float32),
                   pltpu.SemaphoreType.DMA]
    return pl.kernel(body, out_type=out_type, mesh=_mesh,
                     scratch_types=scratch)


def _combine(psum, c0v, c1v):
    BLK = 1000
    G = H // BLK

    def body(p0, p1, c0, c1, o):
        c = c0[...] + c1[...]
        o[...] = (p0[0] + p1[0]) * (1.0 / jnp.maximum(c, 1.0))

    return pl.pallas_call(
        body,
        grid=(G,),
        in_specs=[pl.BlockSpec((1, BLK, D), lambda i: (0, i, 0)),
                  pl.BlockSpec((1, BLK, D), lambda i: (1, i, 0)),
                  pl.BlockSpec((BLK, 1), lambda i: (i, 0)),
                  pl.BlockSpec((BLK, 1), lambda i: (i, 0))],
        out_specs=pl.BlockSpec((BLK, D), lambda i: (i, 0)),
        out_shape=jax.ShapeDtypeStruct((H, D), jnp.float32),
    )(psum, psum, c0v, c1v)


def _final(x, npsum, c0v, c1v, W, b2):
    BLK = 1000
    G = N // BLK

    def body(xr, p0, p1, c0, c1, wr, br, o):
        c = c0[...] + c1[...]
        nf = (p0[0] + p1[0]) * (1.0 / jnp.maximum(c, 1.0))
        w = wr[...]
        out = (jnp.dot(xr[...], w[:D], preferred_element_type=jnp.float32,
                       precision=lax.Precision.HIGHEST)
               + jnp.dot(nf, w[D:], preferred_element_type=jnp.float32,
                         precision=lax.Precision.HIGHEST)
               + br[...])
        s = jnp.sum(out * out, axis=1, keepdims=True)
        o[...] = out / jnp.sqrt(s)

    return pl.pallas_call(
        body,
        grid=(G,),
        in_specs=[pl.BlockSpec((BLK, D), lambda i: (i, 0)),
                  pl.BlockSpec((1, BLK, D), lambda i: (0, i, 0)),
                  pl.BlockSpec((1, BLK, D), lambda i: (1, i, 0)),
                  pl.BlockSpec((BLK, 1), lambda i: (i, 0)),
                  pl.BlockSpec((BLK, 1), lambda i: (i, 0)),
                  pl.BlockSpec((2 * D, D), lambda i: (0, 0)),
                  pl.BlockSpec((1, D), lambda i: (0, 0))],
        out_specs=pl.BlockSpec((BLK, D), lambda i: (i, 0)),
        out_shape=jax.ShapeDtypeStruct((N, D), jnp.float32),
    )(x, npsum, npsum, c0v, c1v, W, b2)


def kernel(x, h_edge_index, W, b):
    he2 = h_edge_index[0].reshape(CHUNKS, K)
    nd2 = h_edge_index[1].reshape(CHUNKS, K)
    zrows = jnp.zeros((H, D), jnp.float32)
    z1d = jnp.zeros((HC,), jnp.float32)
    ones1 = jnp.ones((K,), jnp.float32)

    epsum, hcnt, ncnt = _sc_pass(True)(x, nd2, he2, zrows, z1d, ones1)
    hc0 = hcnt[:H].reshape(H, 1)
    hc1 = hcnt[HC:HC + H].reshape(H, 1)
    nc0 = ncnt[:N].reshape(N, 1)
    nc1 = ncnt[HC:HC + N].reshape(N, 1)
    edge_agg = _combine(epsum, hc0, hc1)
    npsum = _sc_pass(False)(edge_agg, he2, nd2, zrows)
    return _final(x, npsum, nc0, nc1, W, b.reshape(1, D))


# speedup vs baseline: 10.9175x; 1.0027x over previous
import jax
import jax.numpy as jnp
from jax import lax
from jax.experimental import pallas as pl
from jax.experimental.pallas import tpu as pltpu
from jax.experimental.pallas import tpu_sc as plsc

N = 10000
H = 10000
E = 320000
D = 128

NC = 2
NS = 16
NW = NC * NS

K = 125
CHUNKS = E // K
CPW = CHUNKS // NW
SEG = 40
NSEG = CPW // SEG
RPT = 624
TAIL = H - NS * RPT
HC = 10240
CRPT = HC // NS

_mesh = plsc.VectorSubcoreMesh(core_axis_name="c", subcore_axis_name="s")


def _sliced_copy(sid, src_at, dst_at):
    r0 = sid * RPT
    pltpu.sync_copy(src_at(r0, RPT), dst_at(r0, RPT))

    @pl.when(sid == NS - 1)
    def _():
        pltpu.sync_copy(src_at(NS * RPT, TAIL), dst_at(NS * RPT, TAIL))


def _sc_pass(with_counts):
    def body(*refs):
        if with_counts:
            (tbl, gidx2, sidx2, zrows, z1d, ones1,
             psum, scnt_o, gcnt_o,
             acc, scnt, gcnt, gia, sia, rows, ones_v, sem) = refs
        else:
            (tbl, gidx2, sidx2, zrows,
             psum,
             acc, gia, sia, rows, sem) = refs
        cid = lax.axis_index("c")
        sid = lax.axis_index("s")
        wid = sid * NC + cid

        _sliced_copy(sid, lambda r, n: zrows.at[pl.ds(r, n)],
                     lambda r, n: acc.at[pl.ds(r, n)])
        if with_counts:
            w0 = sid * CRPT
            pltpu.sync_copy(z1d.at[pl.ds(w0, CRPT)], scnt.at[pl.ds(w0, CRPT)])
            pltpu.sync_copy(z1d.at[pl.ds(w0, CRPT)], gcnt.at[pl.ds(w0, CRPT)])
            pltpu.sync_copy(ones1, ones_v)
        c0 = wid * CPW
        plsc.subcore_barrier()

        def seg(s, carry):
            sb = c0 + s * SEG
            pltpu.sync_copy(gidx2.at[pl.ds(sb, SEG)], gia)
            pltpu.sync_copy(sidx2.at[pl.ds(sb, SEG)], sia)
            pltpu.async_copy(tbl.at[gia.at[0]], rows.at[0], sem)

            def chunk(i, c2):
                buf = lax.rem(i, 2)

                @pl.when(i + 1 < SEG)
                def _():
                    pltpu.async_copy(tbl.at[gia.at[i + 1]],
                                     rows.at[lax.rem(i + 1, 2)], sem)

                pltpu.make_async_copy(tbl.at[gia.at[i]], rows.at[buf],
                                      sem).wait()
                pltpu.sync_copy(rows.at[buf], acc.at[sia.at[i]], add=True)
                if with_counts:
                    pltpu.sync_copy(ones_v, scnt.at[sia.at[i]], add=True)
                    pltpu.sync_copy(ones_v, gcnt.at[gia.at[i]], add=True)
                return c2

            lax.fori_loop(0, SEG, chunk, 0)
            return carry

        lax.fori_loop(0, NSEG, seg, 0)
        plsc.subcore_barrier()

        _sliced_copy(sid, lambda r, n: acc.at[pl.ds(r, n)],
                     lambda r, n: psum.at[cid, pl.ds(r, n)])
        if with_counts:
            ob = cid * HC + w0
            pltpu.sync_copy(scnt.at[pl.ds(w0, CRPT)],
                            scnt_o.at[pl.ds(ob, CRPT)])
            pltpu.sync_copy(gcnt.at[pl.ds(w0, CRPT)],
                            gcnt_o.at[pl.ds(ob, CRPT)])

    if with_counts:
        out_type = (jax.ShapeDtypeStruct((2, H, D), jnp.float32),
                    jax.ShapeDtypeStruct((2 * HC,), jnp.float32),
                    jax.ShapeDtypeStruct((2 * HC,), jnp.float32))
        scratch = [pltpu.VMEM_SHARED((H, D), jnp.float32),
                   pltpu.VMEM_SHARED((HC,), jnp.float32),
                   pltpu.VMEM_SHARED((HC,), jnp.float32),
                   pltpu.VMEM((SEG, K), jnp.int32),
                   pltpu.VMEM((SEG, K), jnp.int32),
                   pltpu.VMEM((2, K, D), jnp.float32),
                   pltpu.VMEM((K,), jnp.float32),
                   pltpu.SemaphoreType.DMA]
    else:
        out_type = jax.ShapeDtypeStruct((2, H, D), jnp.float32)
        scratch = [pltpu.VMEM_SHARED((H, D), jnp.float32),
                   pltpu.VMEM((SEG, K), jnp.int32),
                   pltpu.VMEM((SEG, K), jnp.int32),
                   pltpu.VMEM((2, K, D), jnp.float32),
                   pltpu.SemaphoreType.DMA]
    return pl.kernel(body, out_type=out_type, mesh=_mesh,
                     scratch_types=scratch)


def _combine(psum, c0v, c1v):
    BLK = 1000
    G = H // BLK

    def body(p0, p1, c0, c1, o):
        c = c0[...] + c1[...]
        o[...] = (p0[0] + p1[0]) * (1.0 / jnp.maximum(c, 1.0))

    return pl.pallas_call(
        body,
        grid=(G,),
        in_specs=[pl.BlockSpec((1, BLK, D), lambda i: (0, i, 0)),
                  pl.BlockSpec((1, BLK, D), lambda i: (1, i, 0)),
                  pl.BlockSpec((BLK, 1), lambda i: (i, 0)),
                  pl.BlockSpec((BLK, 1), lambda i: (i, 0))],
        out_specs=pl.BlockSpec((BLK, D), lambda i: (i, 0)),
        out_shape=jax.ShapeDtypeStruct((H, D), jnp.float32),
    )(psum, psum, c0v, c1v)


def _final(x, npsum, c0v, c1v, W, b2):
    BLK = 1000
    G = N // BLK

    def body(xr, p0, p1, c0, c1, wr, br, o):
        c = c0[...] + c1[...]
        nf = (p0[0] + p1[0]) * (1.0 / jnp.maximum(c, 1.0))
        w = wr[...]
        out = (jnp.dot(xr[...], w[:D], preferred_element_type=jnp.float32,
                       precision=lax.Precision.HIGHEST)
               + jnp.dot(nf, w[D:], preferred_element_type=jnp.float32,
                         precision=lax.Precision.HIGHEST)
               + br[...])
        s = jnp.sum(out * out, axis=1, keepdims=True)
        o[...] = out / jnp.sqrt(s)

    return pl.pallas_call(
        body,
        grid=(G,),
        in_specs=[pl.BlockSpec((BLK, D), lambda i: (i, 0)),
                  pl.BlockSpec((1, BLK, D), lambda i: (0, i, 0)),
                  pl.BlockSpec((1, BLK, D), lambda i: (1, i, 0)),
                  pl.BlockSpec((BLK, 1), lambda i: (i, 0)),
                  pl.BlockSpec((BLK, 1), lambda i: (i, 0)),
                  pl.BlockSpec((2 * D, D), lambda i: (0, 0)),
                  pl.BlockSpec((1, D), lambda i: (0, 0))],
        out_specs=pl.BlockSpec((BLK, D), lambda i: (i, 0)),
        out_shape=jax.ShapeDtypeStruct((N, D), jnp.float32),
    )(x, npsum, npsum, c0v, c1v, W, b2)


def kernel(x, h_edge_index, W, b):
    he2 = h_edge_index[0].reshape(CHUNKS, K)
    nd2 = h_edge_index[1].reshape(CHUNKS, K)
    zrows = jnp.zeros((H, D), jnp.float32)
    z1d = jnp.zeros((HC,), jnp.float32)
    ones1 = jnp.ones((K,), jnp.float32)

    epsum, hcnt, ncnt = _sc_pass(True)(x, nd2, he2, zrows, z1d, ones1)
    hc0 = hcnt[:H].reshape(H, 1)
    hc1 = hcnt[HC:HC + H].reshape(H, 1)
    nc0 = ncnt[:N].reshape(N, 1)
    nc1 = ncnt[HC:HC + N].reshape(N, 1)
    edge_agg = _combine(epsum, hc0, hc1)
    npsum = _sc_pass(False)(edge_agg, he2, nd2, zrows)
    return _final(x, npsum, nc0, nc1, W, b.reshape(1, D))
